# baseline (speedup 1.0000x reference)
import functools

import jax
import jax.numpy as jnp
from jax import lax
from jax.experimental import pallas as pl
from jax.experimental.pallas import tpu as pltpu
from jax.experimental.pallas import tpu_sc as plsc

N = 10000
E = 320000
D = 128
H = 256
C = 64
K = 20

NT = 16
G = 128
R = 3
CH = 162
E_PAD = CH * NT * G
NP = 10240
NPT = NP // NT
NB = 64
NSB = NPT // NB

_f32 = jnp.float32
_i32 = jnp.int32

_MESH = plsc.VectorSubcoreMesh(
    core_axis_name="c", subcore_axis_name="s", num_cores=1, num_subcores=NT)


def _unpack_chunk(packed_v, j, sidx_v, didx_v, b):
  for g in range(G // 16):
    sl = pl.ds(g * 16, 16)
    v = packed_v[j, 0, sl]
    sidx_v[b, sl] = lax.shift_right_logical(v, 16)
    didx_v[b, sl] = lax.bitwise_and(v, 0xFFFF)


def _scale_rows(blk_v, d_v, dbase):
  def body(n16, _):
    dvec = d_v[pl.ds(dbase + n16 * 16, 16)]
    for i in range(16):
      dd = dvec[i]
      row = n16 * 16 + i
      for c4 in range(C // 16):
        sl = pl.ds(c4 * 16, 16)
        blk_v[row, sl] = blk_v[row, sl] * dd
    return 0

  lax.fori_loop(0, NB // 16, body, 0, unroll=False)


def _zero_vmem_2d(ref, rows, cols):
  zero16 = jnp.zeros((16,), _f32)

  def body(i, _):
    for c4 in range(cols // 16):
      ref[i, pl.ds(c4 * 16, 16)] = zero16
    return 0

  lax.fori_loop(0, rows, body, 0, unroll=False)


@functools.partial(
    pl.kernel,
    out_type=(jax.ShapeDtypeStruct((NP,), _f32),
              jax.ShapeDtypeStruct((NP,), _f32)),
    mesh=_MESH,
    scratch_types=[
        pltpu.VMEM_SHARED((NP,), _f32),
        pltpu.VMEM_SHARED((NP,), _f32),
        pltpu.VMEM((CH, 1, G), _i32),
        pltpu.VMEM((1, G), _i32),
        pltpu.VMEM((1, G), _i32),
        pltpu.VMEM((G,), _f32),
        pltpu.VMEM((NPT,), _f32),
    ],
)
def _deg_kernel(packed_hbm, do_hbm, di_hbm,
                do_sh, di_sh, packed_v, sidx_v, didx_v, ones_v, blk_v):
  t = lax.axis_index("s")
  base = t * NPT

  zero16 = jnp.zeros((16,), _f32)
  one16 = jnp.ones((16,), _f32)

  def zb(i, _):
    blk_v[pl.ds(i * 16, 16)] = zero16
    return 0

  lax.fori_loop(0, NPT // 16, zb, 0, unroll=False)
  pltpu.sync_copy(blk_v, do_sh.at[pl.ds(base, NPT)])
  pltpu.sync_copy(blk_v, di_sh.at[pl.ds(base, NPT)])

  pltpu.sync_copy(packed_hbm.at[t], packed_v)
  for g in range(G // 16):
    ones_v[pl.ds(g * 16, 16)] = one16

  plsc.subcore_barrier()

  def chunk(j, _):
    _unpack_chunk(packed_v, j, sidx_v, didx_v, 0)
    pltpu.sync_copy(ones_v, do_sh.at[sidx_v.at[0]], add=True)
    pltpu.sync_copy(ones_v, di_sh.at[didx_v.at[0]], add=True)
    return 0

  lax.fori_loop(0, CH, chunk, 0, unroll=False)
  plsc.subcore_barrier()

  pltpu.sync_copy(do_sh.at[pl.ds(base, NPT)], blk_v)
  pltpu.sync_copy(blk_v, do_hbm.at[pl.ds(base, NPT)])
  pltpu.sync_copy(di_sh.at[pl.ds(base, NPT)], blk_v)
  pltpu.sync_copy(blk_v, di_hbm.at[pl.ds(base, NPT)])


_MLP_BN = 2000


def _mlp_body(x_ref, w1_ref, b1_ref, w2_ref, b2_ref, do_ref, di_ref,
              logits_ref, z0_ref, ddst_ref, dsrc_ref):
  h = jnp.maximum(
      jnp.dot(x_ref[...], w1_ref[...], preferred_element_type=_f32)
      + b1_ref[...], 0.0)
  logits = jnp.dot(h, w2_ref[...], preferred_element_type=_f32) + b2_ref[...]
  dsrc = lax.rsqrt(jnp.maximum(do_ref[...], 1.0))
  ddst = lax.rsqrt(jnp.maximum(di_ref[...], 1.0))
  logits_ref[...] = logits
  z0_ref[...] = logits * dsrc
  ddst_ref[...] = ddst
  dsrc_ref[...] = dsrc


def _mlp_call(x, w1, b1, w2, b2, degout, degin):
  grid = (N // _MLP_BN,)
  return pl.pallas_call(
      _mlp_body,
      grid=grid,
      in_specs=[
          pl.BlockSpec((_MLP_BN, D), lambda i: (i, 0)),
          pl.BlockSpec((D, H), lambda i: (0, 0)),
          pl.BlockSpec((1, H), lambda i: (0, 0)),
          pl.BlockSpec((H, C), lambda i: (0, 0)),
          pl.BlockSpec((1, C), lambda i: (0, 0)),
          pl.BlockSpec((_MLP_BN, 1), lambda i: (i, 0)),
          pl.BlockSpec((_MLP_BN, 1), lambda i: (i, 0)),
      ],
      out_specs=[
          pl.BlockSpec((_MLP_BN, C), lambda i: (i, 0)),
          pl.BlockSpec((_MLP_BN, C), lambda i: (i, 0)),
          pl.BlockSpec((_MLP_BN, 1), lambda i: (i, 0)),
          pl.BlockSpec((_MLP_BN, 1), lambda i: (i, 0)),
      ],
      out_shape=[
          jax.ShapeDtypeStruct((N, C), _f32),
          jax.ShapeDtypeStruct((N, C), _f32),
          jax.ShapeDtypeStruct((N, 1), _f32),
          jax.ShapeDtypeStruct((N, 1), _f32),
      ],
  )(x, w1, b1, w2, b2, degout, degin)


@functools.partial(
    pl.kernel,
    out_type=jax.ShapeDtypeStruct((K, NP, C), _f32),
    mesh=_MESH,
    scratch_types=[
        pltpu.HBM((NP, C), _f32),
        pltpu.VMEM_SHARED((NP, C), _f32),
        pltpu.VMEM((CH, 1, G), _i32),
        pltpu.VMEM((R, G), _i32),
        pltpu.VMEM((R, G), _i32),
        pltpu.VMEM((R, G, C), _f32),
        pltpu.VMEM((NB, C), _f32),
        pltpu.VMEM((NPT,), _f32),
        pltpu.VMEM((NPT,), _f32),
        pltpu.SemaphoreType.DMA((R,)),
        pltpu.SemaphoreType.DMA((R,)),
    ],
)
def _prop_kernel(z0_hbm, ddst_hbm, dsrc_hbm, packed_hbm, out_hbm,
                 z_hbm, acc_sh, packed_v, sidx_v, didx_v, rows_v, blk_v,
                 ddst_v, dsrc_v, gsem, ssem):
  t = lax.axis_index("s")
  base = t * NPT

  pltpu.sync_copy(packed_hbm.at[t], packed_v)
  pltpu.sync_copy(ddst_hbm.at[t], ddst_v)
  pltpu.sync_copy(dsrc_hbm.at[t], dsrc_v)
  for nb in range(NSB):
    sl = pl.ds(base + nb * NB, NB)
    pltpu.sync_copy(z0_hbm.at[sl], blk_v)
    pltpu.sync_copy(blk_v, z_hbm.at[sl])
  _zero_vmem_2d(blk_v, NB, C)
  for nb in range(NSB):
    pltpu.sync_copy(blk_v, acc_sh.at[pl.ds(base + nb * NB, NB)])
  plsc.subcore_barrier()

  def step(k, _):
    def group(m, _):
      @pl.when(m > 0)
      def _():
        for b in range(R):
          pltpu.make_async_copy(
              rows_v.at[b], acc_sh.at[didx_v.at[b]], ssem.at[b]).wait()

      gds = []
      for b in range(R):
        _unpack_chunk(packed_v, m * R + b, sidx_v, didx_v, b)
        gds.append(pltpu.async_copy(
            z_hbm.at[sidx_v.at[b]], rows_v.at[b], gsem.at[b]))
      for b in range(R):
        gds[b].wait()
        pltpu.async_copy(
            rows_v.at[b], acc_sh.at[didx_v.at[b]], ssem.at[b], add=True)
      return 0

    lax.fori_loop(0, CH // R, group, 0, unroll=False)
    for b in range(R):
      pltpu.make_async_copy(
          rows_v.at[b], acc_sh.at[didx_v.at[b]], ssem.at[b]).wait()
    plsc.subcore_barrier()

    for nb in range(NSB):
      sl = pl.ds(base + nb * NB, NB)
      pltpu.sync_copy(acc_sh.at[sl], blk_v)
      _scale_rows(blk_v, ddst_v, nb * NB)
      pltpu.sync_copy(blk_v, out_hbm.at[k, sl])
      _scale_rows(blk_v, dsrc_v, nb * NB)
      pltpu.sync_copy(blk_v, z_hbm.at[sl])
      _zero_vmem_2d(blk_v, NB, C)
      pltpu.sync_copy(blk_v, acc_sh.at[sl])
    plsc.subcore_barrier()
    return 0

  lax.fori_loop(0, K, step, 0, unroll=False)


_GS_BN = 1024


def _gated_body(lg_ref, prop_ref, wg_ref, bg_ref, out_ref):
  wg = wg_ref[...]
  bg = bg_ref[...]
  x0 = lg_ref[...]
  s = jax.nn.sigmoid(jnp.dot(x0, wg, preferred_element_type=_f32) + bg)
  acc = x0 * s
  for k in range(K):
    xk = prop_ref[k]
    s = jax.nn.sigmoid(jnp.dot(xk, wg, preferred_element_type=_f32) + bg)
    acc = acc + xk * s
  out_ref[...] = acc


def _gated_call(logits_p, prop, wg, bg):
  grid = (NP // _GS_BN,)
  return pl.pallas_call(
      _gated_body,
      grid=grid,
      in_specs=[
          pl.BlockSpec((_GS_BN, C), lambda i: (i, 0)),
          pl.BlockSpec((K, _GS_BN, C), lambda i: (0, i, 0)),
          pl.BlockSpec((C, 1), lambda i: (0, 0)),
          pl.BlockSpec((1, 1), lambda i: (0, 0)),
      ],
      out_specs=pl.BlockSpec((_GS_BN, C), lambda i: (i, 0)),
      out_shape=jax.ShapeDtypeStruct((NP, C), _f32),
  )(logits_p, prop, wg, bg)


def kernel(graph, node_features, W1, b1, W2, b2, Wg, bg):
  src = graph[0]
  dst = graph[1]
  pad = E_PAD - E
  padv = jnp.full((pad,), N, _i32)
  srcp = jnp.concatenate([src, padv])
  dstp = jnp.concatenate([dst, padv])
  packed = jnp.bitwise_or(
      jnp.left_shift(srcp, 16), dstp).reshape(NT, CH, 1, G)

  degout_p, degin_p = _deg_kernel(packed)
  degout = degout_p[:N, None]
  degin = degin_p[:N, None]

  logits, z0, ddst, dsrc = _mlp_call(
      node_features, W1, b1.reshape(1, H), W2, b2.reshape(1, C),
      degout, degin)

  zpad = jnp.zeros((NP - N, C), _f32)
  dpad = jnp.zeros((NP - N,), _f32)
  z0p = jnp.concatenate([z0, zpad])
  ddst_p = jnp.concatenate([ddst[:, 0], dpad]).reshape(NT, NPT)
  dsrc_p = jnp.concatenate([dsrc[:, 0], dpad]).reshape(NT, NPT)

  prop = _prop_kernel(z0p, ddst_p, dsrc_p, packed)

  logits_p = jnp.concatenate([logits, zpad])
  out_p = _gated_call(logits_p, prop, Wg, bg.reshape(1, 1))
  return out_p[:N]


# speedup vs baseline: 26.7356x; 26.7356x over previous
import functools

import jax
import jax.numpy as jnp
from jax import lax
from jax.experimental import pallas as pl
from jax.experimental.pallas import tpu as pltpu
from jax.experimental.pallas import tpu_sc as plsc

N = 10000
E = 320000
D = 128
H = 256
C = 64
K = 20

NT = 16
G = 256
CH = 82
CHA = CH + 1
E_PAD = CH * NT * G
E_ALLOC = CHA * NT * G
NP = 10240
NPT = NP // NT
NB = 64
NSB = NPT // NB

_f32 = jnp.float32
_i32 = jnp.int32

_MESH = plsc.VectorSubcoreMesh(
    core_axis_name="c", subcore_axis_name="s", num_cores=1, num_subcores=NT)


def _scale_rows(blk_v, d_v, dbase):
  def body(n16, _):
    dvec = d_v[pl.ds(dbase + n16 * 16, 16)]
    for i in range(16):
      dd = dvec[i]
      row = n16 * 16 + i
      for c4 in range(C // 16):
        sl = pl.ds(c4 * 16, 16)
        blk_v[row, sl] = blk_v[row, sl] * dd
    return 0

  lax.fori_loop(0, NB // 16, body, 0, unroll=False)


def _zero_vmem_2d(ref, rows, cols):
  zero16 = jnp.zeros((16,), _f32)

  def body(i, _):
    for c4 in range(cols // 16):
      ref[i, pl.ds(c4 * 16, 16)] = zero16
    return 0

  lax.fori_loop(0, rows, body, 0, unroll=False)


@functools.partial(
    pl.kernel,
    out_type=(jax.ShapeDtypeStruct((NP,), _f32),
              jax.ShapeDtypeStruct((NP,), _f32)),
    mesh=_MESH,
    scratch_types=[
        pltpu.VMEM_SHARED((NP,), _f32),
        pltpu.VMEM_SHARED((NP,), _f32),
        pltpu.VMEM((CHA, 1, G), _i32),
        pltpu.VMEM((CHA, 1, G), _i32),
        pltpu.VMEM((G,), _f32),
        pltpu.VMEM((NPT,), _f32),
    ],
)
def _deg_kernel(src_hbm, dst_hbm, do_hbm, di_hbm,
                do_sh, di_sh, src_v, dst_v, ones_v, blk_v):
  t = lax.axis_index("s")
  base = t * NPT

  zero16 = jnp.zeros((16,), _f32)
  one16 = jnp.ones((16,), _f32)

  def zb(i, _):
    blk_v[pl.ds(i * 16, 16)] = zero16
    return 0

  lax.fori_loop(0, NPT // 16, zb, 0, unroll=False)
  pltpu.sync_copy(blk_v, do_sh.at[pl.ds(base, NPT)])
  pltpu.sync_copy(blk_v, di_sh.at[pl.ds(base, NPT)])

  pltpu.sync_copy(src_hbm.at[t], src_v)
  pltpu.sync_copy(dst_hbm.at[t], dst_v)
  for g in range(G // 16):
    ones_v[pl.ds(g * 16, 16)] = one16

  plsc.subcore_barrier()

  def chunk(j, _):
    pltpu.sync_copy(ones_v, do_sh.at[src_v.at[j, 0]], add=True)
    pltpu.sync_copy(ones_v, di_sh.at[dst_v.at[j, 0]], add=True)
    return 0

  lax.fori_loop(0, CH, chunk, 0, unroll=False)
  plsc.subcore_barrier()

  pltpu.sync_copy(do_sh.at[pl.ds(base, NPT)], blk_v)
  pltpu.sync_copy(blk_v, do_hbm.at[pl.ds(base, NPT)])
  pltpu.sync_copy(di_sh.at[pl.ds(base, NPT)], blk_v)
  pltpu.sync_copy(blk_v, di_hbm.at[pl.ds(base, NPT)])


_MLP_BN = 2000


def _mlp_body(x_ref, w1_ref, b1_ref, w2_ref, b2_ref, do_ref, di_ref,
              logits_ref, z0_ref, ddst_ref, dsrc_ref):
  h = jnp.maximum(
      jnp.dot(x_ref[...], w1_ref[...], preferred_element_type=_f32)
      + b1_ref[...], 0.0)
  logits = jnp.dot(h, w2_ref[...], preferred_element_type=_f32) + b2_ref[...]
  dsrc = lax.rsqrt(jnp.maximum(do_ref[...], 1.0))
  ddst = lax.rsqrt(jnp.maximum(di_ref[...], 1.0))
  logits_ref[...] = logits
  z0_ref[...] = logits * dsrc
  ddst_ref[...] = ddst
  dsrc_ref[...] = dsrc


def _mlp_call(x, w1, b1, w2, b2, degout, degin):
  grid = (N // _MLP_BN,)
  return pl.pallas_call(
      _mlp_body,
      grid=grid,
      in_specs=[
          pl.BlockSpec((_MLP_BN, D), lambda i: (i, 0)),
          pl.BlockSpec((D, H), lambda i: (0, 0)),
          pl.BlockSpec((1, H), lambda i: (0, 0)),
          pl.BlockSpec((H, C), lambda i: (0, 0)),
          pl.BlockSpec((1, C), lambda i: (0, 0)),
          pl.BlockSpec((_MLP_BN, 1), lambda i: (i, 0)),
          pl.BlockSpec((_MLP_BN, 1), lambda i: (i, 0)),
      ],
      out_specs=[
          pl.BlockSpec((_MLP_BN, C), lambda i: (i, 0)),
          pl.BlockSpec((_MLP_BN, C), lambda i: (i, 0)),
          pl.BlockSpec((_MLP_BN, 1), lambda i: (i, 0)),
          pl.BlockSpec((_MLP_BN, 1), lambda i: (i, 0)),
      ],
      out_shape=[
          jax.ShapeDtypeStruct((N, C), _f32),
          jax.ShapeDtypeStruct((N, C), _f32),
          jax.ShapeDtypeStruct((N, 1), _f32),
          jax.ShapeDtypeStruct((N, 1), _f32),
      ],
  )(x, w1, b1, w2, b2, degout, degin)


@functools.partial(
    pl.kernel,
    out_type=jax.ShapeDtypeStruct((K, NP, C), _f32),
    mesh=_MESH,
    scratch_types=[
        pltpu.VMEM_SHARED((NP, C), _f32),
        pltpu.VMEM_SHARED((NP, C), _f32),
        pltpu.VMEM((2, 1, G), _i32),
        pltpu.VMEM((2, 1, G), _i32),
        pltpu.VMEM((G, C), _f32),
        pltpu.VMEM((NB, C), _f32),
        pltpu.VMEM((NPT,), _f32),
        pltpu.VMEM((NPT,), _f32),
        pltpu.SemaphoreType.DMA((2,)),
        pltpu.SemaphoreType.DMA((2,)),
    ],
)
def _prop_kernel(z0_hbm, ddst_hbm, dsrc_hbm, src_hbm, dst_hbm, out_hbm,
                 z_sh, acc_sh, sidx_v, didx_v, rows_v, blk_v,
                 ddst_v, dsrc_v, isem, jsem):
  t = lax.axis_index("s")
  base = t * NPT

  pltpu.sync_copy(ddst_hbm.at[t], ddst_v)
  pltpu.sync_copy(dsrc_hbm.at[t], dsrc_v)
  for nb in range(NSB):
    sl = pl.ds(base + nb * NB, NB)
    pltpu.sync_copy(z0_hbm.at[sl], blk_v)
    pltpu.sync_copy(blk_v, z_sh.at[sl])
  _zero_vmem_2d(blk_v, NB, C)
  for nb in range(NSB):
    pltpu.sync_copy(blk_v, acc_sh.at[pl.ds(base + nb * NB, NB)])
  plsc.subcore_barrier()

  def _wait_idx(p):
    pltpu.make_async_copy(src_hbm.at[t, 0], sidx_v.at[p], isem.at[p]).wait()
    pltpu.make_async_copy(dst_hbm.at[t, 0], didx_v.at[p], jsem.at[p]).wait()

  def _start_idx(j, p):
    pltpu.async_copy(src_hbm.at[t, j], sidx_v.at[p], isem.at[p])
    pltpu.async_copy(dst_hbm.at[t, j], didx_v.at[p], jsem.at[p])

  def step(k, _):
    _start_idx(0, 0)

    def pair(m, _):
      for p in range(2):
        j = 2 * m + p
        _wait_idx(p)
        _start_idx(j + 1, 1 - p)
        pltpu.sync_copy(z_sh.at[sidx_v.at[p, 0]], rows_v)
        pltpu.sync_copy(rows_v, acc_sh.at[didx_v.at[p, 0]], add=True)
      return 0

    lax.fori_loop(0, CH // 2, pair, 0, unroll=False)
    _wait_idx(0)
    plsc.subcore_barrier()

    for nb in range(NSB):
      sl = pl.ds(base + nb * NB, NB)
      pltpu.sync_copy(acc_sh.at[sl], blk_v)
      _scale_rows(blk_v, ddst_v, nb * NB)
      pltpu.sync_copy(blk_v, out_hbm.at[k, sl])
      _scale_rows(blk_v, dsrc_v, nb * NB)
      pltpu.sync_copy(blk_v, z_sh.at[sl])
      _zero_vmem_2d(blk_v, NB, C)
      pltpu.sync_copy(blk_v, acc_sh.at[sl])
    plsc.subcore_barrier()
    return 0

  lax.fori_loop(0, K, step, 0, unroll=False)


_GS_BN = 1024


def _gated_body(lg_ref, prop_ref, wg_ref, bg_ref, out_ref):
  wg = wg_ref[...]
  bg = bg_ref[...]
  x0 = lg_ref[...]
  s = jax.nn.sigmoid(jnp.dot(x0, wg, preferred_element_type=_f32) + bg)
  acc = x0 * s
  for k in range(K):
    xk = prop_ref[k]
    s = jax.nn.sigmoid(jnp.dot(xk, wg, preferred_element_type=_f32) + bg)
    acc = acc + xk * s
  out_ref[...] = acc


def _gated_call(logits_p, prop, wg, bg):
  grid = (NP // _GS_BN,)
  return pl.pallas_call(
      _gated_body,
      grid=grid,
      in_specs=[
          pl.BlockSpec((_GS_BN, C), lambda i: (i, 0)),
          pl.BlockSpec((K, _GS_BN, C), lambda i: (0, i, 0)),
          pl.BlockSpec((C, 1), lambda i: (0, 0)),
          pl.BlockSpec((1, 1), lambda i: (0, 0)),
      ],
      out_specs=pl.BlockSpec((_GS_BN, C), lambda i: (i, 0)),
      out_shape=jax.ShapeDtypeStruct((NP, C), _f32),
  )(logits_p, prop, wg, bg)


def kernel(graph, node_features, W1, b1, W2, b2, Wg, bg):
  src = graph[0]
  dst = graph[1]
  pad = E_PAD - E
  padv = jnp.full((pad,), N, _i32)
  padc = jnp.full((NT, 1, 1, G), N, _i32)
  srcp = jnp.concatenate(
      [jnp.concatenate([src, padv]).reshape(NT, CH, 1, G), padc], axis=1)
  dstp = jnp.concatenate(
      [jnp.concatenate([dst, padv]).reshape(NT, CH, 1, G), padc], axis=1)

  degout_p, degin_p = _deg_kernel(srcp, dstp)
  degout = degout_p[:N, None]
  degin = degin_p[:N, None]

  logits, z0, ddst, dsrc = _mlp_call(
      node_features, W1, b1.reshape(1, H), W2, b2.reshape(1, C),
      degout, degin)

  zpad = jnp.zeros((NP - N, C), _f32)
  dpad = jnp.zeros((NP - N,), _f32)
  z0p = jnp.concatenate([z0, zpad])
  ddst_p = jnp.concatenate([ddst[:, 0], dpad]).reshape(NT, NPT)
  dsrc_p = jnp.concatenate([dsrc[:, 0], dpad]).reshape(NT, NPT)

  prop = _prop_kernel(z0p, ddst_p, dsrc_p, srcp, dstp)

  logits_p = jnp.concatenate([logits, zpad])
  out_p = _gated_call(logits_p, prop, Wg, bg.reshape(1, 1))
  return out_p[:N]


# speedup vs baseline: 37.7668x; 1.4126x over previous
import functools

import jax
import jax.numpy as jnp
from jax import lax
from jax.experimental import pallas as pl
from jax.experimental.pallas import tpu as pltpu
from jax.experimental.pallas import tpu_sc as plsc

N = 10000
E = 320000
D = 128
H = 256
C = 64
K = 20

NT = 16
G = 128
CHH = 164
CHA = CHH + 2
E_PAD = CHH * NT * G
NP = 10240
NPT = NP // NT
NB = 64
NSB = NPT // NB

_f32 = jnp.float32
_i32 = jnp.int32

_MESH = plsc.VectorSubcoreMesh(
    core_axis_name="c", subcore_axis_name="s", num_cores=1, num_subcores=NT)


def _scale_rows(blk_v, d_v, dbase):
  def body(n16, _):
    dvec = d_v[pl.ds(dbase + n16 * 16, 16)]
    for i in range(16):
      dd = dvec[i]
      row = n16 * 16 + i
      for c4 in range(C // 16):
        sl = pl.ds(c4 * 16, 16)
        blk_v[row, sl] = blk_v[row, sl] * dd
    return 0

  lax.fori_loop(0, NB // 16, body, 0, unroll=False)


def _zero_vmem_2d(ref, rows, cols):
  zero16 = jnp.zeros((16,), _f32)

  def body(i, _):
    for c4 in range(cols // 16):
      ref[i, pl.ds(c4 * 16, 16)] = zero16
    return 0

  lax.fori_loop(0, rows, body, 0, unroll=False)


@functools.partial(
    pl.kernel,
    out_type=(jax.ShapeDtypeStruct((NP,), _f32),
              jax.ShapeDtypeStruct((NP,), _f32)),
    mesh=_MESH,
    scratch_types=[
        pltpu.VMEM_SHARED((NP,), _f32),
        pltpu.VMEM_SHARED((NP,), _f32),
        pltpu.VMEM((CHA, 1, G), _i32),
        pltpu.VMEM((CHA, 1, G), _i32),
        pltpu.VMEM((G,), _f32),
        pltpu.VMEM((NPT,), _f32),
    ],
)
def _deg_kernel(src_hbm, dst_hbm, do_hbm, di_hbm,
                do_sh, di_sh, src_v, dst_v, ones_v, blk_v):
  t = lax.axis_index("s")
  base = t * NPT

  zero16 = jnp.zeros((16,), _f32)
  one16 = jnp.ones((16,), _f32)

  def zb(i, _):
    blk_v[pl.ds(i * 16, 16)] = zero16
    return 0

  lax.fori_loop(0, NPT // 16, zb, 0, unroll=False)
  pltpu.sync_copy(blk_v, do_sh.at[pl.ds(base, NPT)])
  pltpu.sync_copy(blk_v, di_sh.at[pl.ds(base, NPT)])

  pltpu.sync_copy(src_hbm.at[t], src_v)
  pltpu.sync_copy(dst_hbm.at[t], dst_v)
  for g in range(G // 16):
    ones_v[pl.ds(g * 16, 16)] = one16

  plsc.subcore_barrier()

  def chunk(j, _):
    pltpu.sync_copy(ones_v, do_sh.at[src_v.at[j, 0]], add=True)
    pltpu.sync_copy(ones_v, di_sh.at[dst_v.at[j, 0]], add=True)
    return 0

  lax.fori_loop(0, CHH, chunk, 0, unroll=False)
  plsc.subcore_barrier()

  pltpu.sync_copy(do_sh.at[pl.ds(base, NPT)], blk_v)
  pltpu.sync_copy(blk_v, do_hbm.at[pl.ds(base, NPT)])
  pltpu.sync_copy(di_sh.at[pl.ds(base, NPT)], blk_v)
  pltpu.sync_copy(blk_v, di_hbm.at[pl.ds(base, NPT)])


_MLP_BN = 2000


def _mlp_body(x_ref, w1_ref, b1_ref, w2_ref, b2_ref, do_ref, di_ref,
              logits_ref, z0_ref, ddst_ref, dsd_ref):
  h = jnp.maximum(
      jnp.dot(x_ref[...], w1_ref[...], preferred_element_type=_f32)
      + b1_ref[...], 0.0)
  logits = jnp.dot(h, w2_ref[...], preferred_element_type=_f32) + b2_ref[...]
  dsrc = lax.rsqrt(jnp.maximum(do_ref[...], 1.0))
  ddst = lax.rsqrt(jnp.maximum(di_ref[...], 1.0))
  logits_ref[...] = logits
  z0_ref[...] = logits * dsrc
  ddst_ref[...] = ddst
  dsd_ref[...] = dsrc * ddst


def _mlp_call(x, w1, b1, w2, b2, degout, degin):
  grid = (N // _MLP_BN,)
  return pl.pallas_call(
      _mlp_body,
      grid=grid,
      in_specs=[
          pl.BlockSpec((_MLP_BN, D), lambda i: (i, 0)),
          pl.BlockSpec((D, H), lambda i: (0, 0)),
          pl.BlockSpec((1, H), lambda i: (0, 0)),
          pl.BlockSpec((H, C), lambda i: (0, 0)),
          pl.BlockSpec((1, C), lambda i: (0, 0)),
          pl.BlockSpec((_MLP_BN, 1), lambda i: (i, 0)),
          pl.BlockSpec((_MLP_BN, 1), lambda i: (i, 0)),
      ],
      out_specs=[
          pl.BlockSpec((_MLP_BN, C), lambda i: (i, 0)),
          pl.BlockSpec((_MLP_BN, C), lambda i: (i, 0)),
          pl.BlockSpec((_MLP_BN, 1), lambda i: (i, 0)),
          pl.BlockSpec((_MLP_BN, 1), lambda i: (i, 0)),
      ],
      out_shape=[
          jax.ShapeDtypeStruct((N, C), _f32),
          jax.ShapeDtypeStruct((N, C), _f32),
          jax.ShapeDtypeStruct((N, 1), _f32),
          jax.ShapeDtypeStruct((N, 1), _f32),
      ],
  )(x, w1, b1, w2, b2, degout, degin)


@functools.partial(
    pl.kernel,
    out_type=jax.ShapeDtypeStruct((K, NP, C), _f32),
    mesh=_MESH,
    scratch_types=[
        pltpu.VMEM_SHARED((NP, C), _f32),
        pltpu.VMEM_SHARED((NP, C), _f32),
        pltpu.VMEM((4, 1, G), _i32),
        pltpu.VMEM((4, 1, G), _i32),
        pltpu.VMEM((2, G, C), _f32),
        pltpu.VMEM((NB, C), _f32),
        pltpu.VMEM((NPT,), _f32),
        pltpu.SemaphoreType.DMA((4,)),
        pltpu.SemaphoreType.DMA((4,)),
        pltpu.SemaphoreType.DMA((2,)),
        pltpu.SemaphoreType.DMA((2,)),
    ],
)
def _prop_kernel(z0_hbm, dsd_hbm, src_hbm, dst_hbm, out_hbm,
                 z_sh, acc_sh, sidx_v, didx_v, rows_v, blk_v,
                 dsd_v, isem, jsem, gsem, ssem):
  t = lax.axis_index("s")
  base = t * NPT

  pltpu.sync_copy(dsd_hbm.at[t], dsd_v)
  for nb in range(NSB):
    sl = pl.ds(base + nb * NB, NB)
    pltpu.sync_copy(z0_hbm.at[sl], blk_v)
    pltpu.sync_copy(blk_v, z_sh.at[sl])
  _zero_vmem_2d(blk_v, NB, C)
  for nb in range(NSB):
    pltpu.sync_copy(blk_v, acc_sh.at[pl.ds(base + nb * NB, NB)])
  plsc.subcore_barrier()

  def _wait_idx(s):
    pltpu.make_async_copy(src_hbm.at[t, 0], sidx_v.at[s], isem.at[s]).wait()
    pltpu.make_async_copy(dst_hbm.at[t, 0], didx_v.at[s], jsem.at[s]).wait()

  def _start_idx(j, s):
    pltpu.async_copy(src_hbm.at[t, j], sidx_v.at[s], isem.at[s])
    pltpu.async_copy(dst_hbm.at[t, j], didx_v.at[s], jsem.at[s])

  def _wait_scat(p):
    pltpu.make_async_copy(
        rows_v.at[p], acc_sh.at[didx_v.at[p, 0]], ssem.at[p]).wait()

  def step(k, _):
    _start_idx(0, 0)
    _start_idx(1, 1)

    def quad(m, _):
      for q in range(4):
        j = 4 * m + q
        s = q
        p = q % 2
        _wait_idx(s)
        if q < 2:
          @pl.when(m > 0)
          def _():
            _wait_scat(p)
        else:
          _wait_scat(p)
        _start_idx(j + 2, (q + 2) % 4)
        gd = pltpu.async_copy(
            z_sh.at[sidx_v.at[s, 0]], rows_v.at[p], gsem.at[p])
        gd.wait()
        pltpu.async_copy(
            rows_v.at[p], acc_sh.at[didx_v.at[s, 0]], ssem.at[p],
            add=True)
      return 0

    lax.fori_loop(0, CHH // 4, quad, 0, unroll=False)
    _wait_scat(0)
    _wait_scat(1)
    _wait_idx(0)
    _wait_idx(1)
    plsc.subcore_barrier()

    tsl = pl.ds(base, NPT)
    pltpu.sync_copy(acc_sh.at[tsl], out_hbm.at[k, tsl])
    for nb in range(NSB):
      sl = pl.ds(base + nb * NB, NB)
      pltpu.sync_copy(acc_sh.at[sl], blk_v)
      _scale_rows(blk_v, dsd_v, nb * NB)
      pltpu.sync_copy(blk_v, z_sh.at[sl])
    _zero_vmem_2d(blk_v, NB, C)
    for nb in range(NSB):
      pltpu.sync_copy(blk_v, acc_sh.at[pl.ds(base + nb * NB, NB)])
    plsc.subcore_barrier()
    return 0

  lax.fori_loop(0, K, step, 0, unroll=False)


_GS_BN = 1024


def _gated_body(lg_ref, prop_ref, dd_ref, wg_ref, bg_ref, out_ref):
  wg = wg_ref[...]
  bg = bg_ref[...]
  dd = dd_ref[...]
  x0 = lg_ref[...]
  s = jax.nn.sigmoid(jnp.dot(x0, wg, preferred_element_type=_f32) + bg)
  acc = x0 * s
  for k in range(K):
    xk = prop_ref[k] * dd
    s = jax.nn.sigmoid(jnp.dot(xk, wg, preferred_element_type=_f32) + bg)
    acc = acc + xk * s
  out_ref[...] = acc


def _gated_call(logits_p, prop, ddst_p, wg, bg):
  grid = (NP // _GS_BN,)
  return pl.pallas_call(
      _gated_body,
      grid=grid,
      in_specs=[
          pl.BlockSpec((_GS_BN, C), lambda i: (i, 0)),
          pl.BlockSpec((K, _GS_BN, C), lambda i: (0, i, 0)),
          pl.BlockSpec((_GS_BN, 1), lambda i: (i, 0)),
          pl.BlockSpec((C, 1), lambda i: (0, 0)),
          pl.BlockSpec((1, 1), lambda i: (0, 0)),
      ],
      out_specs=pl.BlockSpec((_GS_BN, C), lambda i: (i, 0)),
      out_shape=jax.ShapeDtypeStruct((NP, C), _f32),
  )(logits_p, prop, ddst_p, wg, bg)


def kernel(graph, node_features, W1, b1, W2, b2, Wg, bg):
  src = graph[0]
  dst = graph[1]
  pad = E_PAD - E
  padv = jnp.full((pad,), N, _i32)
  padc = jnp.full((NT, 2, 1, G), N, _i32)
  srcp = jnp.concatenate(
      [jnp.concatenate([src, padv]).reshape(NT, CHH, 1, G), padc], axis=1)
  dstp = jnp.concatenate(
      [jnp.concatenate([dst, padv]).reshape(NT, CHH, 1, G), padc], axis=1)

  degout_p, degin_p = _deg_kernel(srcp, dstp)
  degout = degout_p[:N, None]
  degin = degin_p[:N, None]

  logits, z0, ddst, dsd = _mlp_call(
      node_features, W1, b1.reshape(1, H), W2, b2.reshape(1, C),
      degout, degin)

  zpad = jnp.zeros((NP - N, C), _f32)
  dpad = jnp.zeros((NP - N,), _f32)
  z0p = jnp.concatenate([z0, zpad])
  dsd_p = jnp.concatenate([dsd[:, 0], dpad]).reshape(NT, NPT)
  ddst_p = jnp.concatenate([ddst, jnp.zeros((NP - N, 1), _f32)])

  prop = _prop_kernel(z0p, dsd_p, srcp, dstp)

  logits_p = jnp.concatenate([logits, zpad])
  out_p = _gated_call(logits_p, prop, ddst_p, Wg, bg.reshape(1, 1))
  return out_p[:N]
